# Initial kernel scaffold; baseline (speedup 1.0000x reference)
import jax
import jax.numpy as jnp
from jax.experimental import pallas as pl


def kernel(x, edge_index, fc1_w, fc1_b, conv1_w, conv1_b, conv2_w, conv2_b):
    raise NotImplementedError("write your pallas kernel here")


# speedup vs baseline: 21.0618x; 21.0618x over previous
import functools

import jax
import jax.numpy as jnp
from jax import lax
from jax.experimental import pallas as pl
from jax.experimental.pallas import tpu as pltpu
from jax.experimental.pallas import tpu_sc as plsc

N = 100000
E = 3200000

NC = 2
NS = 16
NW = NC * NS
K = 128

NPAD = 100352
RPS = NPAD // NS
EPAD = 3203072
E_PER_W = EPAD // NW
NBLK = E_PER_W // K

F = 16
FD = 8

_mesh = plsc.VectorSubcoreMesh(core_axis_name="c", subcore_axis_name="s")
_sc_params = pltpu.CompilerParams(use_tc_tiling_on_sc=False)


@functools.partial(
    pl.kernel,
    out_type=jax.ShapeDtypeStruct((NC * NPAD, FD), jnp.float32),
    mesh=_mesh,
    scratch_types=[
        pltpu.VMEM((K,), jnp.int32),
        pltpu.VMEM((K, FD), jnp.float32),
        pltpu.VMEM_SHARED((NPAD, FD), jnp.float32),
    ],
    compiler_params=_sc_params,
)
def _sc_degree(dst_hbm, ones_hbm, zero_hbm, out_hbm, didx, ones_v, acc):
    cid = lax.axis_index("c")
    sid = lax.axis_index("s")
    wid = sid * NC + cid
    r0 = sid * RPS
    pltpu.sync_copy(ones_hbm, ones_v)
    pltpu.sync_copy(zero_hbm.at[pl.ds(r0, RPS)], acc.at[pl.ds(r0, RPS)])
    plsc.subcore_barrier()
    base = wid * E_PER_W

    @pl.loop(0, NBLK)
    def _(b):
        off = base + b * K
        pltpu.sync_copy(dst_hbm.at[pl.ds(off, K)], didx)
        pltpu.sync_copy(ones_v, acc.at[didx], add=True)

    plsc.subcore_barrier()
    pltpu.sync_copy(acc.at[pl.ds(r0, RPS)],
                    out_hbm.at[pl.ds(cid * NPAD + r0, RPS)])


@functools.partial(
    pl.kernel,
    out_type=jax.ShapeDtypeStruct((NC * NPAD, F), jnp.float32),
    mesh=_mesh,
    scratch_types=[
        pltpu.VMEM((K,), jnp.int32),
        pltpu.VMEM((K,), jnp.int32),
        pltpu.VMEM((K, F), jnp.float32),
        pltpu.VMEM_SHARED((NPAD, F), jnp.float32),
        pltpu.SemaphoreType.DMA,
    ],
    compiler_params=_sc_params,
)
def _sc_propagate(g_hbm, src_hbm, dst_hbm, zero_hbm, out_hbm,
                  sidx, didx, rows, acc, sem):
    cid = lax.axis_index("c")
    sid = lax.axis_index("s")
    wid = sid * NC + cid
    r0 = sid * RPS
    pltpu.sync_copy(zero_hbm.at[pl.ds(r0, RPS)], acc.at[pl.ds(r0, RPS)])
    plsc.subcore_barrier()
    base = wid * E_PER_W

    @pl.loop(0, NBLK)
    def _(b):
        off = base + b * K
        pltpu.sync_copy(src_hbm.at[pl.ds(off, K)], sidx)
        pltpu.sync_copy(dst_hbm.at[pl.ds(off, K)], didx)
        pltpu.async_copy(g_hbm.at[sidx], rows, sem).wait()
        pltpu.sync_copy(rows, acc.at[didx], add=True)

    plsc.subcore_barrier()
    pltpu.sync_copy(acc.at[pl.ds(r0, RPS)],
                    out_hbm.at[pl.ds(cid * NPAD + r0, RPS)])


BN = 2048
NB = NPAD // BN


def _tc1_body(x_ref, c0_ref, c1_ref, w1_ref, b1_ref, wc1_ref,
              g1_ref, dinv_ref):
    deg = c0_ref[:, 0:1] + c1_ref[:, 0:1] + 1.0
    dinv = lax.rsqrt(deg)
    h0 = jnp.dot(jnp.trunc(x_ref[...]), w1_ref[...],
                 preferred_element_type=jnp.float32) + b1_ref[...]
    g1_ref[...] = jnp.dot(h0, wc1_ref[...],
                          preferred_element_type=jnp.float32) * dinv
    dinv_ref[...] = jnp.broadcast_to(dinv, (BN, FD))


def _tc1(x_p, cnt, fc1_w, fc1_b, conv1_w):
    return pl.pallas_call(
        _tc1_body,
        grid=(NB,),
        in_specs=[
            pl.BlockSpec((BN, 21), lambda i: (i, 0)),
            pl.BlockSpec((BN, FD), lambda i: (i, 0)),
            pl.BlockSpec((BN, FD), lambda i: (NB + i, 0)),
            pl.BlockSpec((21, 24), lambda i: (0, 0)),
            pl.BlockSpec((1, 24), lambda i: (0, 0)),
            pl.BlockSpec((24, F), lambda i: (0, 0)),
        ],
        out_specs=[
            pl.BlockSpec((BN, F), lambda i: (i, 0)),
            pl.BlockSpec((BN, FD), lambda i: (i, 0)),
        ],
        out_shape=[
            jax.ShapeDtypeStruct((NPAD, F), jnp.float32),
            jax.ShapeDtypeStruct((NPAD, FD), jnp.float32),
        ],
    )(x_p, cnt, cnt, fc1_w, fc1_b.reshape(1, 24), conv1_w)


def _tc2_body(a0_ref, a1_ref, g1_ref, dinv_ref, b1c_ref, w2_ref, g2_ref):
    d = dinv_ref[:, 0:1]
    h1 = d * (a0_ref[...] + a1_ref[...] + g1_ref[...]) + b1c_ref[...]
    h1 = jnp.maximum(h1, 0.0)
    g2_ref[...] = jnp.dot(h1, w2_ref[...],
                          preferred_element_type=jnp.float32) * d


def _tc2(agg1, g1, dinv, conv1_b, conv2_w_pad):
    return pl.pallas_call(
        _tc2_body,
        grid=(NB,),
        in_specs=[
            pl.BlockSpec((BN, F), lambda i: (i, 0)),
            pl.BlockSpec((BN, F), lambda i: (NB + i, 0)),
            pl.BlockSpec((BN, F), lambda i: (i, 0)),
            pl.BlockSpec((BN, FD), lambda i: (i, 0)),
            pl.BlockSpec((1, F), lambda i: (0, 0)),
            pl.BlockSpec((F, F), lambda i: (0, 0)),
        ],
        out_specs=pl.BlockSpec((BN, F), lambda i: (i, 0)),
        out_shape=jax.ShapeDtypeStruct((NPAD, F), jnp.float32),
    )(agg1, agg1, g1, dinv, conv1_b.reshape(1, F), conv2_w_pad)


def _tc3_body(a0_ref, a1_ref, g2_ref, dinv_ref, b2_ref, out_ref):
    d = dinv_ref[:, 0:1]
    o = d * (a0_ref[...] + a1_ref[...] + g2_ref[...]) + b2_ref[...]
    mask = lax.broadcasted_iota(jnp.int32, (1, F), 1) < 6
    z = jnp.where(mask, o, -jnp.inf)
    m = jnp.max(z, axis=1, keepdims=True)
    e = jnp.where(mask, jnp.exp(z - m), 0.0)
    lse = m + jnp.log(jnp.sum(e, axis=1, keepdims=True))
    out_ref[...] = o - lse


def _tc3(agg2, g2, dinv, conv2_b_pad):
    return pl.pallas_call(
        _tc3_body,
        grid=(NB,),
        in_specs=[
            pl.BlockSpec((BN, F), lambda i: (i, 0)),
            pl.BlockSpec((BN, F), lambda i: (NB + i, 0)),
            pl.BlockSpec((BN, F), lambda i: (i, 0)),
            pl.BlockSpec((BN, FD), lambda i: (i, 0)),
            pl.BlockSpec((1, F), lambda i: (0, 0)),
        ],
        out_specs=pl.BlockSpec((BN, F), lambda i: (i, 0)),
        out_shape=jax.ShapeDtypeStruct((NPAD, F), jnp.float32),
    )(agg2, agg2, g2, dinv, conv2_b_pad.reshape(1, F))


def kernel(x, edge_index, fc1_w, fc1_b, conv1_w, conv1_b, conv2_w, conv2_b):
    src = edge_index[0]
    dst = edge_index[1]
    pad_idx = jnp.full((EPAD - E,), NPAD - 1, jnp.int32)
    src_p = jnp.concatenate([src, pad_idx])
    dst_p = jnp.concatenate([dst, pad_idx])
    x_p = jnp.pad(x, ((0, NPAD - N), (0, 0)))
    zeros16 = jnp.zeros((NPAD, F), jnp.float32)
    zeros8 = jnp.zeros((NPAD, FD), jnp.float32)
    ones8 = jnp.ones((K, FD), jnp.float32)
    w2_pad = jnp.pad(conv2_w, ((0, 0), (0, F - 6)))
    b2_pad = jnp.pad(conv2_b, ((0, F - 6),))

    cnt = _sc_degree(dst_p, ones8, zeros8)
    g1, dinv = _tc1(x_p, cnt, fc1_w, fc1_b, conv1_w)
    agg1 = _sc_propagate(g1, src_p, dst_p, zeros16)
    g2 = _tc2(agg1, g1, dinv, conv1_b, w2_pad)
    agg2 = _sc_propagate(g2, src_p, dst_p, zeros16)
    out = _tc3(agg2, g2, dinv, b2_pad)
    return out[:N, :6]


# speedup vs baseline: 52.0693x; 2.4722x over previous
import functools

import jax
import jax.numpy as jnp
from jax import lax
from jax.experimental import pallas as pl
from jax.experimental.pallas import tpu as pltpu
from jax.experimental.pallas import tpu_sc as plsc

N = 100000
E = 3200000

NC = 2
NS = 16
NW = NC * NS
K = 128
C = 8

NPAD = 100352
RPS = NPAD // NS
NBLK = 784
EPAD = NW * NBLK * K
TOTBLK = EPAD // K
NCHUNK = NBLK // C

F = 16
FD = 8
F2 = 8

_mesh = plsc.VectorSubcoreMesh(core_axis_name="c", subcore_axis_name="s")
_sc_params = pltpu.CompilerParams(use_tc_tiling_on_sc=False)


@functools.partial(
    pl.kernel,
    out_type=jax.ShapeDtypeStruct((NC * NPAD, FD), jnp.float32),
    mesh=_mesh,
    scratch_types=[
        pltpu.VMEM((2, C, K), jnp.int32),
        pltpu.VMEM((K, FD), jnp.float32),
        pltpu.VMEM_SHARED((NPAD, FD), jnp.float32),
        pltpu.SemaphoreType.DMA,
    ],
    compiler_params=_sc_params,
)
def _sc_degree(dst_hbm, ones_hbm, zero_hbm, out_hbm, didx, ones_v, acc, isem):
    cid = lax.axis_index("c")
    sid = lax.axis_index("s")
    wid = sid * NC + cid
    r0 = sid * RPS
    pltpu.sync_copy(ones_hbm, ones_v)
    pltpu.sync_copy(zero_hbm.at[pl.ds(r0, RPS)], acc.at[pl.ds(r0, RPS)])
    plsc.subcore_barrier()

    bb = wid * NBLK
    pltpu.sync_copy(dst_hbm.at[pl.ds(bb, C)], didx.at[0])

    @pl.loop(0, NCHUNK // 2)
    def _(half):
        for p in range(2):
            c = half * 2 + p

            @pl.when(c < NCHUNK - 1)
            def _():
                nb = bb + (c + 1) * C
                pltpu.async_copy(dst_hbm.at[pl.ds(nb, C)], didx.at[1 - p], isem)

            for j in range(C):
                pltpu.sync_copy(ones_v, acc.at[didx.at[p, j]], add=True)

            @pl.when(c < NCHUNK - 1)
            def _():
                pltpu.make_async_copy(
                    dst_hbm.at[pl.ds(bb, C)], didx.at[1 - p], isem).wait()

    plsc.subcore_barrier()
    pltpu.sync_copy(acc.at[pl.ds(r0, RPS)],
                    out_hbm.at[pl.ds(cid * NPAD + r0, RPS)])


def _make_propagate(FP):
    @functools.partial(
        pl.kernel,
        out_type=jax.ShapeDtypeStruct((NC * NPAD, FP), jnp.float32),
        mesh=_mesh,
        scratch_types=[
            pltpu.VMEM((2, C, K), jnp.int32),
            pltpu.VMEM((2, C, K), jnp.int32),
            pltpu.VMEM((K, FP), jnp.float32),
            pltpu.VMEM((K, FP), jnp.float32),
            pltpu.VMEM_SHARED((NPAD, FP), jnp.float32),
            pltpu.SemaphoreType.DMA,
            pltpu.SemaphoreType.DMA,
        ],
        compiler_params=_sc_params,
    )
    def _prop(g_hbm, src_hbm, dst_hbm, zero_hbm, out_hbm,
              sidx, didx, rows0, rows1, acc, gsem, isem):
        cid = lax.axis_index("c")
        sid = lax.axis_index("s")
        wid = sid * NC + cid
        r0 = sid * RPS
        pltpu.sync_copy(zero_hbm.at[pl.ds(r0, RPS)], acc.at[pl.ds(r0, RPS)])
        plsc.subcore_barrier()

        bb = wid * NBLK
        pltpu.sync_copy(src_hbm.at[pl.ds(bb, C)], sidx.at[0])
        pltpu.sync_copy(dst_hbm.at[pl.ds(bb, C)], didx.at[0])
        pltpu.async_copy(g_hbm.at[sidx.at[0, 0]], rows0, gsem)
        rbufs = (rows0, rows1)

        @pl.loop(0, NCHUNK // 2)
        def _(half):
            for p in range(2):
                c = half * 2 + p

                @pl.when(c < NCHUNK - 1)
                def _():
                    nb = bb + (c + 1) * C
                    pltpu.async_copy(src_hbm.at[pl.ds(nb, C)], sidx.at[1 - p],
                                     isem)
                    pltpu.async_copy(dst_hbm.at[pl.ds(nb, C)], didx.at[1 - p],
                                     isem)

                for j in range(C):
                    cur = rbufs[j % 2]
                    nxt = rbufs[(j + 1) % 2]
                    if j < C - 1:
                        pltpu.async_copy(g_hbm.at[sidx.at[p, j + 1]], nxt,
                                         gsem)
                    else:
                        @pl.when(c < NCHUNK - 1)
                        def _():
                            pltpu.make_async_copy(
                                src_hbm.at[pl.ds(bb, C)], sidx.at[1 - p],
                                isem).wait()
                            pltpu.make_async_copy(
                                dst_hbm.at[pl.ds(bb, C)], didx.at[1 - p],
                                isem).wait()
                            pltpu.async_copy(g_hbm.at[sidx.at[1 - p, 0]], nxt,
                                             gsem)
                    pltpu.make_async_copy(g_hbm.at[sidx.at[p, j]], cur,
                                          gsem).wait()
                    pltpu.sync_copy(cur, acc.at[didx.at[p, j]], add=True)

        plsc.subcore_barrier()
        pltpu.sync_copy(acc.at[pl.ds(r0, RPS)],
                        out_hbm.at[pl.ds(cid * NPAD + r0, RPS)])

    return _prop


_sc_prop16 = _make_propagate(F)
_sc_prop8 = _make_propagate(F2)


BN = 2048
NB = NPAD // BN


def _tc1_body(x_ref, c0_ref, c1_ref, w1_ref, b1_ref, wc1_ref,
              g1_ref, dinv_ref):
    deg = c0_ref[:, 0:1] + c1_ref[:, 0:1] + 1.0
    dinv = lax.rsqrt(deg)
    h0 = jnp.dot(jnp.trunc(x_ref[...]), w1_ref[...],
                 preferred_element_type=jnp.float32) + b1_ref[...]
    g1_ref[...] = jnp.dot(h0, wc1_ref[...],
                          preferred_element_type=jnp.float32) * dinv
    dinv_ref[...] = jnp.broadcast_to(dinv, (BN, FD))


def _tc1(x_p, cnt, fc1_w, fc1_b, conv1_w):
    return pl.pallas_call(
        _tc1_body,
        grid=(NB,),
        in_specs=[
            pl.BlockSpec((BN, 21), lambda i: (i, 0)),
            pl.BlockSpec((BN, FD), lambda i: (i, 0)),
            pl.BlockSpec((BN, FD), lambda i: (NB + i, 0)),
            pl.BlockSpec((21, 24), lambda i: (0, 0)),
            pl.BlockSpec((1, 24), lambda i: (0, 0)),
            pl.BlockSpec((24, F), lambda i: (0, 0)),
        ],
        out_specs=[
            pl.BlockSpec((BN, F), lambda i: (i, 0)),
            pl.BlockSpec((BN, FD), lambda i: (i, 0)),
        ],
        out_shape=[
            jax.ShapeDtypeStruct((NPAD, F), jnp.float32),
            jax.ShapeDtypeStruct((NPAD, FD), jnp.float32),
        ],
    )(x_p, cnt, cnt, fc1_w, fc1_b.reshape(1, 24), conv1_w)


def _tc2_body(a0_ref, a1_ref, g1_ref, dinv_ref, b1c_ref, w2_ref, g2_ref):
    d = dinv_ref[:, 0:1]
    h1 = d * (a0_ref[...] + a1_ref[...] + g1_ref[...]) + b1c_ref[...]
    h1 = jnp.maximum(h1, 0.0)
    g2_ref[...] = jnp.dot(h1, w2_ref[...],
                          preferred_element_type=jnp.float32) * d


def _tc2(agg1, g1, dinv, conv1_b, conv2_w_pad):
    return pl.pallas_call(
        _tc2_body,
        grid=(NB,),
        in_specs=[
            pl.BlockSpec((BN, F), lambda i: (i, 0)),
            pl.BlockSpec((BN, F), lambda i: (NB + i, 0)),
            pl.BlockSpec((BN, F), lambda i: (i, 0)),
            pl.BlockSpec((BN, FD), lambda i: (i, 0)),
            pl.BlockSpec((1, F), lambda i: (0, 0)),
            pl.BlockSpec((F, F2), lambda i: (0, 0)),
        ],
        out_specs=pl.BlockSpec((BN, F2), lambda i: (i, 0)),
        out_shape=jax.ShapeDtypeStruct((NPAD, F2), jnp.float32),
    )(agg1, agg1, g1, dinv, conv1_b.reshape(1, F), conv2_w_pad)


def _tc3_body(a0_ref, a1_ref, g2_ref, dinv_ref, b2_ref, out_ref):
    d = dinv_ref[:, 0:1]
    o = d * (a0_ref[...] + a1_ref[...] + g2_ref[...]) + b2_ref[...]
    mask = lax.broadcasted_iota(jnp.int32, (1, F2), 1) < 6
    z = jnp.where(mask, o, -jnp.inf)
    m = jnp.max(z, axis=1, keepdims=True)
    e = jnp.where(mask, jnp.exp(z - m), 0.0)
    lse = m + jnp.log(jnp.sum(e, axis=1, keepdims=True))
    out_ref[...] = o - lse


def _tc3(agg2, g2, dinv, conv2_b_pad):
    return pl.pallas_call(
        _tc3_body,
        grid=(NB,),
        in_specs=[
            pl.BlockSpec((BN, F2), lambda i: (i, 0)),
            pl.BlockSpec((BN, F2), lambda i: (NB + i, 0)),
            pl.BlockSpec((BN, F2), lambda i: (i, 0)),
            pl.BlockSpec((BN, FD), lambda i: (i, 0)),
            pl.BlockSpec((1, F2), lambda i: (0, 0)),
        ],
        out_specs=pl.BlockSpec((BN, F2), lambda i: (i, 0)),
        out_shape=jax.ShapeDtypeStruct((NPAD, F2), jnp.float32),
    )(agg2, agg2, g2, dinv, conv2_b_pad.reshape(1, F2))


def kernel(x, edge_index, fc1_w, fc1_b, conv1_w, conv1_b, conv2_w, conv2_b):
    src = edge_index[0]
    dst = edge_index[1]
    pad_idx = jnp.full((EPAD - E,), NPAD - 1, jnp.int32)
    src_p = jnp.concatenate([src, pad_idx]).reshape(TOTBLK, K)
    dst_p = jnp.concatenate([dst, pad_idx]).reshape(TOTBLK, K)
    x_p = jnp.pad(x, ((0, NPAD - N), (0, 0)))
    zeros16 = jnp.zeros((NPAD, F), jnp.float32)
    zeros8 = jnp.zeros((NPAD, FD), jnp.float32)
    ones8 = jnp.ones((K, FD), jnp.float32)
    w2_pad = jnp.pad(conv2_w, ((0, 0), (0, F2 - 6)))
    b2_pad = jnp.pad(conv2_b, ((0, F2 - 6),))

    cnt = _sc_degree(dst_p, ones8, zeros8)
    g1, dinv = _tc1(x_p, cnt, fc1_w, fc1_b, conv1_w)
    agg1 = _sc_prop16(g1, src_p, dst_p, zeros16)
    g2 = _tc2(agg1, g1, dinv, conv1_b, w2_pad)
    agg2 = _sc_prop8(g2, src_p, dst_p, zeros8)
    out = _tc3(agg2, g2, dinv, b2_pad)
    return out[:N, :6]


# speedup vs baseline: 83.6709x; 1.6069x over previous
import functools

import jax
import jax.numpy as jnp
from jax import lax
from jax.experimental import pallas as pl
from jax.experimental.pallas import tpu as pltpu
from jax.experimental.pallas import tpu_sc as plsc

N = 100000
E = 3200000

NC = 2
NS = 16
NW = NC * NS
K = 128
C = 8
NRB = 4

NPAD = 100352
RPS = NPAD // NS
NBLK = 784
EPAD = NW * NBLK * K
TOTBLK = EPAD // K
NCHUNK = NBLK // C

F = 16
NP = NPAD // 8

_mesh = plsc.VectorSubcoreMesh(core_axis_name="c", subcore_axis_name="s")
_sc_params = pltpu.CompilerParams(use_tc_tiling_on_sc=False)


@functools.partial(
    pl.kernel,
    out_type=jax.ShapeDtypeStruct((NC * NPAD, F), jnp.float32),
    mesh=_mesh,
    scratch_types=[
        pltpu.VMEM((2, C, K), jnp.int32),
        pltpu.VMEM((K, F), jnp.float32),
        pltpu.VMEM_SHARED((NPAD, F), jnp.float32),
        pltpu.SemaphoreType.DMA,
    ],
    compiler_params=_sc_params,
)
def _sc_degree(dst_hbm, ones_hbm, zero_hbm, out_hbm, didx, ones_v, acc, isem):
    cid = lax.axis_index("c")
    sid = lax.axis_index("s")
    wid = sid * NC + cid
    r0 = sid * RPS
    pltpu.sync_copy(ones_hbm, ones_v)
    pltpu.sync_copy(zero_hbm.at[pl.ds(r0, RPS)], acc.at[pl.ds(r0, RPS)])
    plsc.subcore_barrier()

    bb = wid * NBLK
    pltpu.sync_copy(dst_hbm.at[pl.ds(bb, C)], didx.at[0])

    @pl.loop(0, NCHUNK // 2)
    def _(half):
        for p in range(2):
            c = half * 2 + p

            @pl.when(c < NCHUNK - 1)
            def _():
                nb = bb + (c + 1) * C
                pltpu.async_copy(dst_hbm.at[pl.ds(nb, C)], didx.at[1 - p], isem)

            for j in range(C):
                pltpu.sync_copy(ones_v, acc.at[didx.at[p, j]], add=True)

            @pl.when(c < NCHUNK - 1)
            def _():
                pltpu.make_async_copy(
                    dst_hbm.at[pl.ds(bb, C)], didx.at[1 - p], isem).wait()

    plsc.subcore_barrier()
    pltpu.sync_copy(acc.at[pl.ds(r0, RPS)],
                    out_hbm.at[pl.ds(cid * NPAD + r0, RPS)])


@functools.partial(
    pl.kernel,
    out_type=jax.ShapeDtypeStruct((NC * NPAD, F), jnp.float32),
    mesh=_mesh,
    scratch_types=[
        pltpu.VMEM((2, C, K), jnp.int32),
        pltpu.VMEM((2, C, K), jnp.int32),
    ] + [pltpu.VMEM((K, F), jnp.float32)] * NRB + [
        pltpu.VMEM_SHARED((NPAD, F), jnp.float32),
        pltpu.SemaphoreType.DMA,
        pltpu.SemaphoreType.DMA,
    ],
    compiler_params=_sc_params,
)
def _sc_propagate(g_hbm, src_hbm, dst_hbm, zero_hbm, out_hbm,
                  sidx, didx, rows0, rows1, rows2, rows3, acc, gsem, isem):
    cid = lax.axis_index("c")
    sid = lax.axis_index("s")
    wid = sid * NC + cid
    r0 = sid * RPS
    pltpu.sync_copy(zero_hbm.at[pl.ds(r0, RPS)], acc.at[pl.ds(r0, RPS)])
    plsc.subcore_barrier()

    bb = wid * NBLK
    pltpu.sync_copy(src_hbm.at[pl.ds(bb, C)], sidx.at[0])
    pltpu.sync_copy(dst_hbm.at[pl.ds(bb, C)], didx.at[0])
    rbufs = (rows0, rows1, rows2, rows3)
    for j in range(NRB - 1):
        pltpu.async_copy(g_hbm.at[sidx.at[0, j]], rbufs[j], gsem)

    @pl.loop(0, NCHUNK // 2)
    def _(half):
        for p in range(2):
            c = half * 2 + p

            @pl.when(c < NCHUNK - 1)
            def _():
                nb = bb + (c + 1) * C
                pltpu.async_copy(src_hbm.at[pl.ds(nb, C)], sidx.at[1 - p],
                                 isem)
                pltpu.async_copy(dst_hbm.at[pl.ds(nb, C)], didx.at[1 - p],
                                 isem)

            for j in range(C):
                ahead = j + NRB - 1
                abuf = rbufs[ahead % NRB]
                if ahead < C:
                    pltpu.async_copy(g_hbm.at[sidx.at[p, ahead]], abuf, gsem)
                else:
                    @pl.when(c < NCHUNK - 1)
                    def _():
                        if ahead == C:
                            pltpu.make_async_copy(
                                src_hbm.at[pl.ds(bb, C)], sidx.at[1 - p],
                                isem).wait()
                            pltpu.make_async_copy(
                                dst_hbm.at[pl.ds(bb, C)], didx.at[1 - p],
                                isem).wait()
                        pltpu.async_copy(
                            g_hbm.at[sidx.at[1 - p, ahead - C]], abuf, gsem)
                cur = rbufs[j % NRB]
                pltpu.make_async_copy(g_hbm.at[sidx.at[p, j]], cur,
                                      gsem).wait()
                pltpu.sync_copy(cur, acc.at[didx.at[p, j]], add=True)

    plsc.subcore_barrier()
    pltpu.sync_copy(acc.at[pl.ds(r0, RPS)],
                    out_hbm.at[pl.ds(cid * NPAD + r0, RPS)])


NB = 7
BP = NP // NB
XW = 8 * 21


def _tc1_body(x_ref, c0_ref, c1_ref, bdw1_ref, b1t_ref, bdwc1_ref,
              g1_ref, dinv_ref):
    dinv = lax.rsqrt(c0_ref[...] + c1_ref[...] + 1.0)
    bdm1 = jnp.dot(bdw1_ref[...], bdwc1_ref[...],
                   preferred_element_type=jnp.float32)
    c1t = jnp.dot(b1t_ref[...], bdwc1_ref[...],
                  preferred_element_type=jnp.float32)
    g1_ref[...] = (jnp.dot(jnp.trunc(x_ref[...]), bdm1,
                           preferred_element_type=jnp.float32) + c1t) * dinv
    dinv_ref[...] = dinv


def _tc1(x_pk, cntp, bd_fc1w, fc1b_t, bd_w1):
    return pl.pallas_call(
        _tc1_body,
        grid=(NB,),
        in_specs=[
            pl.BlockSpec((BP, XW), lambda i: (i, 0)),
            pl.BlockSpec((BP, 128), lambda i: (i, 0)),
            pl.BlockSpec((BP, 128), lambda i: (NB + i, 0)),
            pl.BlockSpec((XW, 8 * 24), lambda i: (0, 0)),
            pl.BlockSpec((1, 8 * 24), lambda i: (0, 0)),
            pl.BlockSpec((8 * 24, 128), lambda i: (0, 0)),
        ],
        out_specs=[
            pl.BlockSpec((BP, 128), lambda i: (i, 0)),
            pl.BlockSpec((BP, 128), lambda i: (i, 0)),
        ],
        out_shape=[
            jax.ShapeDtypeStruct((NP, 128), jnp.float32),
            jax.ShapeDtypeStruct((NP, 128), jnp.float32),
        ],
    )(x_pk, cntp, cntp, bd_fc1w, fc1b_t, bd_w1)


def _tc2_body(a0_ref, a1_ref, g1_ref, dinv_ref, b1t_ref, bdw2_ref, g2_ref):
    d = dinv_ref[...]
    h1 = d * (a0_ref[...] + a1_ref[...] + g1_ref[...]) + b1t_ref[...]
    h1 = jnp.maximum(h1, 0.0)
    g2_ref[...] = jnp.dot(h1, bdw2_ref[...],
                          preferred_element_type=jnp.float32) * d


def _tc2(agg1p, g1p, dinvp, b1_t, bd_w2):
    return pl.pallas_call(
        _tc2_body,
        grid=(NB,),
        in_specs=[
            pl.BlockSpec((BP, 128), lambda i: (i, 0)),
            pl.BlockSpec((BP, 128), lambda i: (NB + i, 0)),
            pl.BlockSpec((BP, 128), lambda i: (i, 0)),
            pl.BlockSpec((BP, 128), lambda i: (i, 0)),
            pl.BlockSpec((1, 128), lambda i: (0, 0)),
            pl.BlockSpec((128, 128), lambda i: (0, 0)),
        ],
        out_specs=pl.BlockSpec((BP, 128), lambda i: (i, 0)),
        out_shape=jax.ShapeDtypeStruct((NP, 128), jnp.float32),
    )(agg1p, agg1p, g1p, dinvp, b1_t, bd_w2)


def _tc3_body(a0_ref, a1_ref, g2_ref, dinv_ref, b2t_ref, out_ref):
    o = dinv_ref[...] * (a0_ref[...] + a1_ref[...] + g2_ref[...]) + b2t_ref[...]
    mask = lax.broadcasted_iota(jnp.int32, (1, 128), 1) % F < 6
    z = jnp.where(mask, o, -jnp.inf)
    m = jnp.concatenate(
        [jnp.broadcast_to(
            jnp.max(z[:, i * F:(i + 1) * F], axis=1, keepdims=True), (BP, F))
         for i in range(8)], axis=1)
    e = jnp.where(mask, jnp.exp(z - m), 0.0)
    s = jnp.concatenate(
        [jnp.broadcast_to(
            jnp.sum(e[:, i * F:(i + 1) * F], axis=1, keepdims=True), (BP, F))
         for i in range(8)], axis=1)
    out_ref[...] = o - (m + jnp.log(s))


def _tc3(agg2p, g2p, dinvp, b2_t):
    return pl.pallas_call(
        _tc3_body,
        grid=(NB,),
        in_specs=[
            pl.BlockSpec((BP, 128), lambda i: (i, 0)),
            pl.BlockSpec((BP, 128), lambda i: (NB + i, 0)),
            pl.BlockSpec((BP, 128), lambda i: (i, 0)),
            pl.BlockSpec((BP, 128), lambda i: (i, 0)),
            pl.BlockSpec((1, 128), lambda i: (0, 0)),
        ],
        out_specs=pl.BlockSpec((BP, 128), lambda i: (i, 0)),
        out_shape=jax.ShapeDtypeStruct((NP, 128), jnp.float32),
    )(agg2p, agg2p, g2p, dinvp, b2_t)


def kernel(x, edge_index, fc1_w, fc1_b, conv1_w, conv1_b, conv2_w, conv2_b):
    src = edge_index[0]
    dst = edge_index[1]
    pad_idx = jnp.full((EPAD - E,), NPAD - 1, jnp.int32)
    src_p = jnp.concatenate([src, pad_idx]).reshape(TOTBLK, K)
    dst_p = jnp.concatenate([dst, pad_idx]).reshape(TOTBLK, K)
    x_pk = jnp.pad(x, ((0, NPAD - N), (0, 0))).reshape(NP, XW)
    zeros16 = jnp.zeros((NPAD, F), jnp.float32)
    ones16 = jnp.ones((K, F), jnp.float32)
    eye8 = jnp.eye(8, dtype=jnp.float32)
    bd_fc1w = jnp.kron(eye8, fc1_w)
    bd_w1 = jnp.kron(eye8, conv1_w)
    w2_pad = jnp.pad(conv2_w, ((0, 0), (0, F - 6)))
    bd_w2 = jnp.kron(eye8, w2_pad)
    fc1b_t = jnp.tile(fc1_b, 8).reshape(1, 8 * 24)
    b1_t = jnp.tile(conv1_b, 8).reshape(1, 128)
    b2_t = jnp.tile(jnp.pad(conv2_b, ((0, F - 6),)), 8).reshape(1, 128)

    cnt = _sc_degree(dst_p, ones16, zeros16)
    cntp = jnp.reshape(cnt, (2 * NP, 128))
    g1p, dinvp = _tc1(x_pk, cntp, bd_fc1w, fc1b_t, bd_w1)
    agg1 = _sc_propagate(jnp.reshape(g1p, (NPAD, F)), src_p, dst_p, zeros16)
    agg1p = jnp.reshape(agg1, (2 * NP, 128))
    g2p = _tc2(agg1p, g1p, dinvp, b1_t, bd_w2)
    agg2 = _sc_propagate(jnp.reshape(g2p, (NPAD, F)), src_p, dst_p, zeros16)
    agg2p = jnp.reshape(agg2, (2 * NP, 128))
    outp = _tc3(agg2p, g2p, dinvp, b2_t)
    return jnp.reshape(outp, (NPAD, F))[:N, :6]
